# baseline (speedup 1.0000x reference)
import functools

import jax
import jax.numpy as jnp
from jax import lax
from jax.experimental import pallas as pl
from jax.experimental.pallas import tpu as pltpu
from jax.experimental.pallas import tpu_sc as plsc

_CAM_F = 248.0
_CAM_C = 111.5

_NUM_CORES = 2
_NUM_SUBCORES = 16
_NUM_TILES = _NUM_CORES * _NUM_SUBCORES
_K = 16
_CHUNKS = 20
_PTS = _K * _CHUNKS

_LEVELS = ((56, 56, 96), (28, 28, 192), (14, 14, 384), (7, 7, 768))
_COL_OFF = (3, 99, 291, 675)
_D_OUT = 3 + sum(c for _, _, c in _LEVELS)

_BCAST_DNUMS = lax.GatherDimensionNumbers(
    offset_dims=(), collapsed_slice_dims=(0,), start_index_map=(0,))


def _lane_bcast(vec, p):
    return lax.gather(vec, jnp.full((_K, 1), p, jnp.int32), _BCAST_DNUMS,
                      slice_sizes=(1,),
                      mode=lax.GatherScatterMode.PROMISE_IN_BOUNDS)


def _floor_f32(x):
    t = x.astype(jnp.int32)
    tf = t.astype(jnp.float32)
    return t - (x < tf).astype(jnp.int32)


def _sc_body(t0, t1, t2, t3, xyz, out, *scr):
    sts = scr[0:16]
    ibuf = scr[16:18]
    wbuf = scr[18:20]
    xv, yv, zv, outbuf = scr[20:24]
    glsem = scr[24:28]
    osem, xsem = scr[28:30]
    tabs = (t0, t1, t2, t3)

    B = xyz.shape[0]
    N = xyz.shape[2]

    cid_ax = lax.axis_index("c")
    sid_ax = lax.axis_index("s")
    wid = sid_ax * _NUM_CORES + cid_ax
    pstart = jnp.minimum(wid * _PTS, N - _PTS)

    lanes = lax.iota(jnp.int32, _K)
    lrow = lanes * _D_OUT

    def level_args(l, buf):
        return [(tabs[l].at[ibuf[buf].at[l * 4 + k]], sts[l * 4 + k],
                 glsem[l]) for k in range(4)]

    def fire_level(l, buf):
        for a in level_args(l, buf):
            pltpu.async_copy(*a)

    def wait_level(l, buf):
        for a in level_args(l, buf):
            pltpu.make_async_copy(*a).wait()

    def prep_idx(ci, buf, b):
        sl = pl.ds(ci * _K, _K)
        x = xv[sl]
        y = yv[sl]
        z = zv[sl]
        w = jnp.clip(-_CAM_F * (x / z) / _CAM_C, -1.0, 1.0)
        h = jnp.clip(_CAM_F * (y / z) / _CAM_C, -1.0, 1.0)
        for l, (H, W, C) in enumerate(_LEVELS):
            ix = ((w + 1.0) * W - 1.0) * 0.5
            iy = ((h + 1.0) * H - 1.0) * 0.5
            ix0 = _floor_f32(ix)
            iy0 = _floor_f32(iy)
            wx1 = ix - ix0.astype(jnp.float32)
            wy1 = iy - iy0.astype(jnp.float32)
            wx0 = (1.0 - wx1) * (ix0 >= 0).astype(jnp.float32)
            wx1 = wx1 * (ix0 <= W - 2).astype(jnp.float32)
            wy0 = (1.0 - wy1) * (iy0 >= 0).astype(jnp.float32)
            wy1 = wy1 * (iy0 <= H - 2).astype(jnp.float32)
            cx0 = jnp.maximum(ix0, 0)
            cx1 = jnp.minimum(ix0 + 1, W - 1)
            by0 = jnp.maximum(iy0, 0) * W + b * (H * W)
            by1 = jnp.minimum(iy0 + 1, H - 1) * W + b * (H * W)
            ibuf[buf][l * 4 + 0, :] = by0 + cx0
            ibuf[buf][l * 4 + 1, :] = by0 + cx1
            ibuf[buf][l * 4 + 2, :] = by1 + cx0
            ibuf[buf][l * 4 + 3, :] = by1 + cx1
            wbuf[buf][l * 4 + 0, :] = wx0 * wy0
            wbuf[buf][l * 4 + 1, :] = wx1 * wy0
            wbuf[buf][l * 4 + 2, :] = wx0 * wy1
            wbuf[buf][l * 4 + 3, :] = wx1 * wy1

    def compute_level(l, buf):
        _, _, C = _LEVELS[l]
        st0 = sts[l * 4 + 0]
        st1 = sts[l * 4 + 1]
        st2 = sts[l * 4 + 2]
        st3 = sts[l * 4 + 3]
        wr0 = wbuf[buf][l * 4 + 0, :]
        wr1 = wbuf[buf][l * 4 + 1, :]
        wr2 = wbuf[buf][l * 4 + 2, :]
        wr3 = wbuf[buf][l * 4 + 3, :]
        off = _COL_OFF[l]

        def pbody(p, _):
            w00 = _lane_bcast(wr0, p)
            w01 = _lane_bcast(wr1, p)
            w10 = _lane_bcast(wr2, p)
            w11 = _lane_bcast(wr3, p)
            obase = p * _D_OUT + off
            for cb in range(C // _K):
                csl = pl.ds(cb * _K, _K)
                acc = (st0[p, csl] * w00 + st1[p, csl] * w01
                       + st2[p, csl] * w10 + st3[p, csl] * w11)
                outbuf[pl.ds(obase + cb * _K, _K)] = acc
            return 0

        lax.fori_loop(0, _K, pbody, 0)

    def do_chunk(ci, cur, nxt, b, has_next, not_first):
        @pl.when(has_next)
        def _():
            prep_idx(ci + 1, nxt, b)

        @pl.when(not_first)
        def _():
            pltpu.make_async_copy(
                outbuf, out.at[pl.ds(0, _K * _D_OUT)], osem).wait()

        sl = pl.ds(ci * _K, _K)
        plsc.store_scatter(outbuf, [lrow], xv[sl])
        plsc.store_scatter(outbuf, [lrow + 1], yv[sl])
        plsc.store_scatter(outbuf, [lrow + 2], zv[sl])
        for l in range(4):
            wait_level(l, cur)
            compute_level(l, cur)

            @pl.when(has_next)
            def _(l=l, nxt=nxt):
                fire_level(l, nxt)

        p0 = pstart + ci * _K
        pltpu.async_copy(
            outbuf, out.at[pl.ds((b * N + p0) * _D_OUT, _K * _D_OUT)], osem)

    def batch_body(b, carry):
        dxs = [pltpu.async_copy(xyz.at[b, j, pl.ds(pstart, _PTS)], v, xsem)
               for j, v in enumerate((xv, yv, zv))]
        for d in dxs:
            d.wait()
        prep_idx(0, 0, b)
        for l in range(4):
            fire_level(l, 0)

        true_ = jnp.bool_(True)

        def pair_body(i, c):
            do_chunk(2 * i, 0, 1, b, true_, (b > 0) | (i > 0))
            do_chunk(2 * i + 1, 1, 0, b, i < _CHUNKS // 2 - 1, true_)
            return c

        lax.fori_loop(0, _CHUNKS // 2, pair_body, 0)
        return carry

    lax.fori_loop(0, B, batch_body, 0)
    pltpu.make_async_copy(
        outbuf, out.at[pl.ds(0, _K * _D_OUT)], osem).wait()


def _make_sc_call(B, N):
    scratch = []
    for _, _, C in _LEVELS:
        scratch.extend([pltpu.VMEM((_K, C), jnp.float32)] * 4)
    scratch += [pltpu.VMEM((_K, _K), jnp.int32)] * 2
    scratch += [pltpu.VMEM((_K, _K), jnp.float32)] * 2
    scratch += [
        pltpu.VMEM((_PTS,), jnp.float32),
        pltpu.VMEM((_PTS,), jnp.float32),
        pltpu.VMEM((_PTS,), jnp.float32),
        pltpu.VMEM((_K * _D_OUT,), jnp.float32),
        pltpu.SemaphoreType.DMA,
        pltpu.SemaphoreType.DMA,
        pltpu.SemaphoreType.DMA,
        pltpu.SemaphoreType.DMA,
        pltpu.SemaphoreType.DMA,
        pltpu.SemaphoreType.DMA,
    ]
    mesh = plsc.VectorSubcoreMesh(
        core_axis_name="c", subcore_axis_name="s",
        num_cores=_NUM_CORES, num_subcores=_NUM_SUBCORES)
    return pl.kernel(
        _sc_body,
        out_type=jax.ShapeDtypeStruct((B * N * _D_OUT,), jnp.float32),
        mesh=mesh,
        scratch_types=scratch,
        compiler_params=pltpu.CompilerParams(
            use_tc_tiling_on_sc=False, needs_layout_passes=False),
    )


def _transpose_call(Bn, C, HW):
    def body(in_ref, out_ref):
        out_ref[0] = in_ref[0].T

    return pl.pallas_call(
        body,
        grid=(Bn,),
        in_specs=[pl.BlockSpec((1, C, HW), lambda b: (b, 0, 0))],
        out_specs=pl.BlockSpec((1, HW, C), lambda b: (b, 0, 0)),
        out_shape=jax.ShapeDtypeStruct((Bn, HW, C), jnp.float32),
    )


def _make_table(f):
    B, C, H, W = f.shape
    fr = f.reshape(B, C, H * W)
    return _transpose_call(B, C, H * W)(fr).reshape(B * H * W, C)


@jax.jit
def kernel(img_feat_0, img_feat_1, img_feat_2, img_feat_3, inputs):
    B, N, _ = inputs.shape
    tables = [_make_table(f)
              for f in (img_feat_0, img_feat_1, img_feat_2, img_feat_3)]
    xyz = jnp.transpose(inputs, (0, 2, 1))
    flat = _make_sc_call(B, N)(*tables, xyz)
    return flat.reshape(B, N, _D_OUT)


# speedup vs baseline: 1.2799x; 1.2799x over previous
import functools

import jax
import jax.numpy as jnp
from jax import lax
from jax.experimental import pallas as pl
from jax.experimental.pallas import tpu as pltpu
from jax.experimental.pallas import tpu_sc as plsc

_CAM_F = 248.0
_CAM_C = 111.5

_NUM_CORES = 2
_NUM_SUBCORES = 16
_NUM_TILES = _NUM_CORES * _NUM_SUBCORES
_K = 16
_CHUNKS = 20
_PTS = _K * _CHUNKS

_LEVELS = ((56, 56, 96), (28, 28, 192), (14, 14, 384), (7, 7, 768))
_COL_OFF = (3, 99, 291, 675)
_D_OUT = 3 + sum(c for _, _, c in _LEVELS)

_BCAST_DNUMS = lax.GatherDimensionNumbers(
    offset_dims=(), collapsed_slice_dims=(0,), start_index_map=(0,))


def _lane_bcast(vec, p):
    return lax.gather(vec, jnp.full((_K, 1), p, jnp.int32), _BCAST_DNUMS,
                      slice_sizes=(1,),
                      mode=lax.GatherScatterMode.PROMISE_IN_BOUNDS)


def _floor_f32(x):
    t = x.astype(jnp.int32)
    tf = t.astype(jnp.float32)
    return t - (x < tf).astype(jnp.int32)


def _sc_body(t0, t1, t2, t3, xyz, out, *scr):
    sts = scr[0:16]
    ibuf = scr[16:18]
    wbuf = scr[18:20]
    xyzv, outbuf = scr[20:22]
    glsem = scr[22:26]
    osem, xsem = scr[26:28]
    tabs = (t0, t1, t2, t3)

    B = xyz.shape[0]
    N = xyz.shape[1] // 3

    cid_ax = lax.axis_index("c")
    sid_ax = lax.axis_index("s")
    wid = sid_ax * _NUM_CORES + cid_ax
    pstart = jnp.minimum(wid * _PTS, N - _PTS)

    lanes = lax.iota(jnp.int32, _K)
    lrow = lanes * _D_OUT
    lxyz = lanes * 3

    def level_args(l, buf):
        return [(tabs[l].at[ibuf[buf].at[l * 4 + k]], sts[l * 4 + k],
                 glsem[l]) for k in range(4)]

    def fire_level(l, buf):
        for a in level_args(l, buf):
            pltpu.async_copy(*a)

    def wait_level(l, buf):
        for a in level_args(l, buf):
            pltpu.make_async_copy(*a).wait()

    def load_xyz(ci):
        g = lxyz + ci * (3 * _K)
        x = plsc.load_gather(xyzv, [g])
        y = plsc.load_gather(xyzv, [g + 1])
        z = plsc.load_gather(xyzv, [g + 2])
        return x, y, z

    def prep_idx(ci, buf, b):
        x, y, z = load_xyz(ci)
        w = jnp.clip(-_CAM_F * (x / z) / _CAM_C, -1.0, 1.0)
        h = jnp.clip(_CAM_F * (y / z) / _CAM_C, -1.0, 1.0)
        for l, (H, W, C) in enumerate(_LEVELS):
            Wp = W + 2
            Vp = (H + 2) * Wp
            ix = ((w + 1.0) * W - 1.0) * 0.5
            iy = ((h + 1.0) * H - 1.0) * 0.5
            ix0 = _floor_f32(ix)
            iy0 = _floor_f32(iy)
            wx1 = ix - ix0.astype(jnp.float32)
            wy1 = iy - iy0.astype(jnp.float32)
            wx0 = 1.0 - wx1
            wy0 = 1.0 - wy1
            base = (iy0 + 1) * Wp + (ix0 + 1) + b * Vp
            ibuf[buf][l * 4 + 0, :] = base
            ibuf[buf][l * 4 + 1, :] = base + 1
            ibuf[buf][l * 4 + 2, :] = base + Wp
            ibuf[buf][l * 4 + 3, :] = base + Wp + 1
            wbuf[buf][l * 4 + 0, :] = wx0 * wy0
            wbuf[buf][l * 4 + 1, :] = wx1 * wy0
            wbuf[buf][l * 4 + 2, :] = wx0 * wy1
            wbuf[buf][l * 4 + 3, :] = wx1 * wy1

    def compute_level(l, buf):
        _, _, C = _LEVELS[l]
        st0 = sts[l * 4 + 0]
        st1 = sts[l * 4 + 1]
        st2 = sts[l * 4 + 2]
        st3 = sts[l * 4 + 3]
        wr0 = wbuf[buf][l * 4 + 0, :]
        wr1 = wbuf[buf][l * 4 + 1, :]
        wr2 = wbuf[buf][l * 4 + 2, :]
        wr3 = wbuf[buf][l * 4 + 3, :]
        off = _COL_OFF[l]

        def pbody(p, _):
            w00 = _lane_bcast(wr0, p)
            w01 = _lane_bcast(wr1, p)
            w10 = _lane_bcast(wr2, p)
            w11 = _lane_bcast(wr3, p)
            obase = p * _D_OUT + off
            for cb in range(C // _K):
                csl = pl.ds(cb * _K, _K)
                acc = (st0[p, csl] * w00 + st1[p, csl] * w01
                       + st2[p, csl] * w10 + st3[p, csl] * w11)
                outbuf[pl.ds(obase + cb * _K, _K)] = acc
            return 0

        lax.fori_loop(0, _K, pbody, 0)

    def do_chunk(ci, cur, nxt, b, has_next, not_first):
        @pl.when(has_next)
        def _():
            prep_idx(ci + 1, nxt, b)

        @pl.when(not_first)
        def _():
            pltpu.make_async_copy(
                outbuf, out.at[pl.ds(0, _K * _D_OUT)], osem).wait()

        x, y, z = load_xyz(ci)
        plsc.store_scatter(outbuf, [lrow], x)
        plsc.store_scatter(outbuf, [lrow + 1], y)
        plsc.store_scatter(outbuf, [lrow + 2], z)
        for l in range(4):
            wait_level(l, cur)
            compute_level(l, cur)

            @pl.when(has_next)
            def _(l=l, nxt=nxt):
                fire_level(l, nxt)

        p0 = pstart + ci * _K
        pltpu.async_copy(
            outbuf, out.at[pl.ds((b * N + p0) * _D_OUT, _K * _D_OUT)], osem)

    def batch_body(b, carry):
        pltpu.async_copy(
            xyz.at[b, pl.ds(pstart * 3, _PTS * 3)], xyzv, xsem).wait()
        prep_idx(0, 0, b)
        for l in range(4):
            fire_level(l, 0)

        true_ = jnp.bool_(True)

        def pair_body(i, c):
            do_chunk(2 * i, 0, 1, b, true_, (b > 0) | (i > 0))
            do_chunk(2 * i + 1, 1, 0, b, i < _CHUNKS // 2 - 1, true_)
            return c

        lax.fori_loop(0, _CHUNKS // 2, pair_body, 0)
        return carry

    lax.fori_loop(0, B, batch_body, 0)
    pltpu.make_async_copy(
        outbuf, out.at[pl.ds(0, _K * _D_OUT)], osem).wait()


def _make_sc_call(B, N):
    scratch = []
    for _, _, C in _LEVELS:
        scratch.extend([pltpu.VMEM((_K, C), jnp.float32)] * 4)
    scratch += [pltpu.VMEM((_K, _K), jnp.int32)] * 2
    scratch += [pltpu.VMEM((_K, _K), jnp.float32)] * 2
    scratch += [
        pltpu.VMEM((_PTS * 3,), jnp.float32),
        pltpu.VMEM((_K * _D_OUT,), jnp.float32),
        pltpu.SemaphoreType.DMA,
        pltpu.SemaphoreType.DMA,
        pltpu.SemaphoreType.DMA,
        pltpu.SemaphoreType.DMA,
        pltpu.SemaphoreType.DMA,
        pltpu.SemaphoreType.DMA,
    ]
    mesh = plsc.VectorSubcoreMesh(
        core_axis_name="c", subcore_axis_name="s",
        num_cores=_NUM_CORES, num_subcores=_NUM_SUBCORES)
    return pl.kernel(
        _sc_body,
        out_type=jax.ShapeDtypeStruct((B * N * _D_OUT,), jnp.float32),
        mesh=mesh,
        scratch_types=scratch,
        compiler_params=pltpu.CompilerParams(
            use_tc_tiling_on_sc=False, needs_layout_passes=False),
    )


def _transpose_call(Bn, C, HW):
    def body(in_ref, out_ref):
        out_ref[0] = in_ref[0].T

    return pl.pallas_call(
        body,
        grid=(Bn,),
        in_specs=[pl.BlockSpec((1, C, HW), lambda b: (b, 0, 0))],
        out_specs=pl.BlockSpec((1, HW, C), lambda b: (b, 0, 0)),
        out_shape=jax.ShapeDtypeStruct((Bn, HW, C), jnp.float32),
    )


def _make_table(f):
    B, C, H, W = f.shape
    fp = jnp.pad(f, ((0, 0), (0, 0), (1, 1), (1, 1)))
    hw = (H + 2) * (W + 2)
    fr = fp.reshape(B, C, hw)
    return _transpose_call(B, C, hw)(fr).reshape(B * hw, C)


@jax.jit
def kernel(img_feat_0, img_feat_1, img_feat_2, img_feat_3, inputs):
    B, N, _ = inputs.shape
    tables = [_make_table(f)
              for f in (img_feat_0, img_feat_1, img_feat_2, img_feat_3)]
    xyz = inputs.reshape(B, N * 3)
    flat = _make_sc_call(B, N)(*tables, xyz)
    return flat.reshape(B, N, _D_OUT)


# speedup vs baseline: 1.3436x; 1.0497x over previous
import functools

import jax
import jax.numpy as jnp
from jax import lax
from jax.experimental import pallas as pl
from jax.experimental.pallas import tpu as pltpu
from jax.experimental.pallas import tpu_sc as plsc

_CAM_F = 248.0
_CAM_C = 111.5

_NUM_CORES = 2
_NUM_SUBCORES = 16
_NUM_TILES = _NUM_CORES * _NUM_SUBCORES
_K = 16
_CHUNKS = 20
_PTS = _K * _CHUNKS

_LEVELS = ((56, 56, 96), (28, 28, 192), (14, 14, 384), (7, 7, 768))
_COL_OFF = (3, 99, 291, 675)
_D_OUT = 3 + sum(c for _, _, c in _LEVELS)

_BCAST_DNUMS = lax.GatherDimensionNumbers(
    offset_dims=(), collapsed_slice_dims=(0,), start_index_map=(0,))


def _lane_bcast(vec, p):
    return lax.gather(vec, jnp.full((_K, 1), p, jnp.int32), _BCAST_DNUMS,
                      slice_sizes=(1,),
                      mode=lax.GatherScatterMode.PROMISE_IN_BOUNDS)


def _floor_f32(x):
    t = x.astype(jnp.int32)
    tf = t.astype(jnp.float32)
    return t - (x < tf).astype(jnp.int32)


def _sc_body(t0, t1, t2, t3, xyz, out, *scr):
    sts = scr[0:16]
    ibuf = scr[16:18]
    wbuf = scr[18:20]
    xyzv, outbuf = scr[20:22]
    glsem = scr[22:26]
    osem, xsem = scr[26:28]
    tabs = (t0, t1, t2, t3)

    B = xyz.shape[0]
    N = xyz.shape[1] // 3

    cid_ax = lax.axis_index("c")
    sid_ax = lax.axis_index("s")
    wid = sid_ax * _NUM_CORES + cid_ax
    pstart = jnp.minimum(wid * _PTS, N - _PTS)

    lanes = lax.iota(jnp.int32, _K)
    lxyz = lanes * 3

    def level_args(l, buf):
        return [(tabs[l].at[ibuf[buf].at[l * 4 + k]], sts[l * 4 + k],
                 glsem[l]) for k in range(4)]

    def fire_level(l, buf):
        for a in level_args(l, buf):
            pltpu.async_copy(*a)

    def wait_level(l, buf):
        for a in level_args(l, buf):
            pltpu.make_async_copy(*a).wait()

    def load_xyz(ci):
        g = lxyz + ci * (3 * _K)
        x = plsc.load_gather(xyzv, [g])
        y = plsc.load_gather(xyzv, [g + 1])
        z = plsc.load_gather(xyzv, [g + 2])
        return x, y, z

    def prep_idx(ci, buf, b):
        x, y, z = load_xyz(ci)
        w = jnp.clip(-_CAM_F * (x / z) / _CAM_C, -1.0, 1.0)
        h = jnp.clip(_CAM_F * (y / z) / _CAM_C, -1.0, 1.0)
        for l, (H, W, C) in enumerate(_LEVELS):
            Wp = W + 2
            Vp = (H + 2) * Wp
            ix = ((w + 1.0) * W - 1.0) * 0.5
            iy = ((h + 1.0) * H - 1.0) * 0.5
            ix0 = _floor_f32(ix)
            iy0 = _floor_f32(iy)
            wx1 = ix - ix0.astype(jnp.float32)
            wy1 = iy - iy0.astype(jnp.float32)
            wx0 = 1.0 - wx1
            wy0 = 1.0 - wy1
            base = (iy0 + 1) * Wp + (ix0 + 1) + b * Vp
            ibuf[buf][l * 4 + 0, :] = base
            ibuf[buf][l * 4 + 1, :] = base + 1
            ibuf[buf][l * 4 + 2, :] = base + Wp
            ibuf[buf][l * 4 + 3, :] = base + Wp + 1
            wbuf[buf][l * 4 + 0, :] = wx0 * wy0
            wbuf[buf][l * 4 + 1, :] = wx1 * wy0
            wbuf[buf][l * 4 + 2, :] = wx0 * wy1
            wbuf[buf][l * 4 + 3, :] = wx1 * wy1

    def compute_level(l, buf):
        _, _, C = _LEVELS[l]
        st0 = sts[l * 4 + 0]
        st1 = sts[l * 4 + 1]
        st2 = sts[l * 4 + 2]
        st3 = sts[l * 4 + 3]
        wr0 = wbuf[buf][l * 4 + 0, :]
        wr1 = wbuf[buf][l * 4 + 1, :]
        wr2 = wbuf[buf][l * 4 + 2, :]
        wr3 = wbuf[buf][l * 4 + 3, :]
        off = _COL_OFF[l]

        def pbody(p, _):
            w00 = _lane_bcast(wr0, p)
            w01 = _lane_bcast(wr1, p)
            w10 = _lane_bcast(wr2, p)
            w11 = _lane_bcast(wr3, p)
            for cb in range(C // _K):
                csl = pl.ds(cb * _K, _K)
                acc = (st0[p, csl] * w00 + st1[p, csl] * w01
                       + st2[p, csl] * w10 + st3[p, csl] * w11)
                outbuf[p, pl.ds(off + cb * _K, _K)] = acc
            return 0

        lax.fori_loop(0, _K, pbody, 0)

    def do_chunk(ci, cur, nxt, b, has_next, not_first):
        @pl.when(has_next)
        def _():
            prep_idx(ci + 1, nxt, b)

        @pl.when(not_first)
        def _():
            pltpu.make_async_copy(
                outbuf, out.at[0, pl.ds(0, _K), :], osem).wait()

        x, y, z = load_xyz(ci)
        plsc.store_scatter(outbuf, [lanes, jnp.full((_K,), 0, jnp.int32)], x)
        plsc.store_scatter(outbuf, [lanes, jnp.full((_K,), 1, jnp.int32)], y)
        plsc.store_scatter(outbuf, [lanes, jnp.full((_K,), 2, jnp.int32)], z)
        for l in range(4):
            wait_level(l, cur)
            compute_level(l, cur)

            @pl.when(has_next)
            def _(l=l, nxt=nxt):
                fire_level(l, nxt)

        p0 = pstart + ci * _K
        pltpu.async_copy(outbuf, out.at[b, pl.ds(p0, _K), :], osem)

    def batch_body(b, carry):
        pltpu.async_copy(
            xyz.at[b, pl.ds(pstart * 3, _PTS * 3)], xyzv, xsem).wait()
        prep_idx(0, 0, b)
        for l in range(4):
            fire_level(l, 0)

        true_ = jnp.bool_(True)

        def pair_body(i, c):
            do_chunk(2 * i, 0, 1, b, true_, (b > 0) | (i > 0))
            do_chunk(2 * i + 1, 1, 0, b, i < _CHUNKS // 2 - 1, true_)
            return c

        lax.fori_loop(0, _CHUNKS // 2, pair_body, 0)
        return carry

    lax.fori_loop(0, B, batch_body, 0)
    pltpu.make_async_copy(
        outbuf, out.at[0, pl.ds(0, _K), :], osem).wait()


def _make_sc_call(B, N):
    scratch = []
    for _, _, C in _LEVELS:
        scratch.extend([pltpu.VMEM((_K, C), jnp.float32)] * 4)
    scratch += [pltpu.VMEM((_K, _K), jnp.int32)] * 2
    scratch += [pltpu.VMEM((_K, _K), jnp.float32)] * 2
    scratch += [
        pltpu.VMEM((_PTS * 3,), jnp.float32),
        pltpu.VMEM((_K, _D_OUT), jnp.float32),
        pltpu.SemaphoreType.DMA,
        pltpu.SemaphoreType.DMA,
        pltpu.SemaphoreType.DMA,
        pltpu.SemaphoreType.DMA,
        pltpu.SemaphoreType.DMA,
        pltpu.SemaphoreType.DMA,
    ]
    mesh = plsc.VectorSubcoreMesh(
        core_axis_name="c", subcore_axis_name="s",
        num_cores=_NUM_CORES, num_subcores=_NUM_SUBCORES)
    return pl.kernel(
        _sc_body,
        out_type=jax.ShapeDtypeStruct((B, N, _D_OUT), jnp.float32),
        mesh=mesh,
        scratch_types=scratch,
        compiler_params=pltpu.CompilerParams(
            use_tc_tiling_on_sc=False, needs_layout_passes=False),
    )


def _transpose_call(Bn, C, HW):
    def body(in_ref, out_ref):
        out_ref[0] = in_ref[0].T

    return pl.pallas_call(
        body,
        grid=(Bn,),
        in_specs=[pl.BlockSpec((1, C, HW), lambda b: (b, 0, 0))],
        out_specs=pl.BlockSpec((1, HW, C), lambda b: (b, 0, 0)),
        out_shape=jax.ShapeDtypeStruct((Bn, HW, C), jnp.float32),
    )


def _make_table(f):
    B, C, H, W = f.shape
    fp = jnp.pad(f, ((0, 0), (0, 0), (1, 1), (1, 1)))
    hw = (H + 2) * (W + 2)
    fr = fp.reshape(B, C, hw)
    return _transpose_call(B, C, hw)(fr).reshape(B * hw, C)


@jax.jit
def kernel(img_feat_0, img_feat_1, img_feat_2, img_feat_3, inputs):
    B, N, _ = inputs.shape
    tables = [_make_table(f)
              for f in (img_feat_0, img_feat_1, img_feat_2, img_feat_3)]
    xyz = inputs.reshape(B, N * 3)
    return _make_sc_call(B, N)(*tables, xyz)
